# baseline (speedup 1.0000x reference)
import functools

import jax
import jax.numpy as jnp
from jax import lax
from jax.experimental import pallas as pl
from jax.experimental.pallas import tpu as pltpu, tpu_sc as plsc

INPUT_DIM = 100000
D = 32
B = 16384
S = 200

NC = 2
NS = 16
NW = NC * NS
G = 8
PER_W = B // NW
CHUNKS = PER_W // G


def _embed_body(x_hbm, tok_hbm, pos_hbm, out_hbm,
                idx0, idx1, rows0, rows1, pos_v,
                gsem0, gsem1, osem0, osem1, isem0, isem1):
    wid = lax.axis_index("s") * NC + lax.axis_index("c")
    base = wid * PER_W

    pltpu.sync_copy(pos_hbm.at[pl.ds(0, S)], pos_v)

    bufs = ((idx0, rows0, gsem0, osem0, isem0),
            (idx1, rows1, gsem1, osem1, isem1))

    def idx_src(g):
        return x_hbm.at[pl.ds(base + g * G, G), :]

    def out_dst(g):
        return out_hbm.at[pl.ds(base + g * G, G)]

    def fire_gather(idx_v, rows_v, sem):
        for s in range(G):
            pltpu.async_copy(tok_hbm.at[idx_v.at[s]], rows_v.at[s], sem)

    def wait_gather(idx_v, rows_v, sem):
        for s in range(G):
            pltpu.make_async_copy(tok_hbm.at[idx_v.at[s]], rows_v.at[s], sem).wait()

    def add_pos(rows_v):
        @pl.loop(0, S)
        def _add(r):
            p0 = pos_v[r, pl.ds(0, 16)]
            p1 = pos_v[r, pl.ds(16, 16)]
            for s in range(G):
                plsc.addupdate(rows_v.at[s, r, pl.ds(0, 16)], p0)
                plsc.addupdate(rows_v.at[s, r, pl.ds(16, 16)], p1)

    pltpu.sync_copy(idx_src(0), idx0)
    pltpu.async_copy(idx_src(1), idx1, isem1)
    fire_gather(idx0, rows0, gsem0)

    @pl.loop(0, CHUNKS // 2)
    def _pair(k):
        for b in range(2):
            g = 2 * k + b
            c_idx, c_rows, c_gsem, c_osem, c_isem = bufs[b]
            n_idx, n_rows, n_gsem, n_osem, n_isem = bufs[1 - b]

            @pl.when(g + 1 < CHUNKS)
            def _fire_next():
                @pl.when(g >= 1)
                def _wait_prev_scatter():
                    pltpu.make_async_copy(n_rows, out_dst(g - 1), n_osem).wait()
                pltpu.make_async_copy(idx_src(g + 1), n_idx, n_isem).wait()
                fire_gather(n_idx, n_rows, n_gsem)

            wait_gather(c_idx, c_rows, c_gsem)

            @pl.when(g + 2 < CHUNKS)
            def _prefetch_idx():
                pltpu.async_copy(idx_src(g + 2), c_idx, c_isem)

            add_pos(c_rows)
            pltpu.async_copy(c_rows, out_dst(g), c_osem)

    pltpu.make_async_copy(rows0, out_dst(CHUNKS - 2), osem0).wait()
    pltpu.make_async_copy(rows1, out_dst(CHUNKS - 1), osem1).wait()


@jax.jit
def _embed(x2, token_table, position_table):
    mesh = plsc.VectorSubcoreMesh(core_axis_name="c", subcore_axis_name="s")
    return pl.kernel(
        _embed_body,
        out_type=jax.ShapeDtypeStruct((B, S, D), jnp.float32),
        mesh=mesh,
        compiler_params=pltpu.CompilerParams(
            use_tc_tiling_on_sc=False, needs_layout_passes=False),
        scratch_types=[
            pltpu.VMEM((G, S), jnp.int32),
            pltpu.VMEM((G, S), jnp.int32),
            pltpu.VMEM((G, S, D), jnp.float32),
            pltpu.VMEM((G, S, D), jnp.float32),
            pltpu.VMEM((S, D), jnp.float32),
            pltpu.SemaphoreType.DMA,
            pltpu.SemaphoreType.DMA,
            pltpu.SemaphoreType.DMA,
            pltpu.SemaphoreType.DMA,
            pltpu.SemaphoreType.DMA,
            pltpu.SemaphoreType.DMA,
        ],
    )(x2, token_table, position_table)


def kernel(x, token_table, position_table):
    return _embed(x.astype(jnp.int32), token_table, position_table)


# speedup vs baseline: 1.9148x; 1.9148x over previous
import functools

import jax
import jax.numpy as jnp
from jax import lax
from jax.experimental import pallas as pl
from jax.experimental.pallas import tpu as pltpu, tpu_sc as plsc

INPUT_DIM = 100000
D = 32
B = 16384
S = 200

NC = 2
NS = 16
NW = NC * NS
BW = B // NW
BWP = 515


def _embed_body(xt_hbm, tok_hbm, pos_hbm, out_hbm,
                idx0, idx1, rows0, rows1, tb0, tb1, pos_v,
                isem0, isem1, gsem0, gsem1, osem0, osem1):
    wid = lax.axis_index("s") * NC + lax.axis_index("c")
    b0 = wid * BW

    pltpu.sync_copy(pos_hbm, pos_v)

    iota = lax.iota(jnp.int32, 16)
    dhi = iota + 16

    bufs = ((idx0, rows0, tb0, isem0, gsem0, osem0),
            (idx1, rows1, tb1, isem1, gsem1, osem1))

    def idx_src(l):
        return xt_hbm.at[l, pl.ds(b0, BW)]

    def out_dst(l):
        return out_hbm.at[l, :, pl.ds(b0, BW)]

    def transpose_add(l, rows_v, tb_v):
        plo = pos_v[pl.ds(l * D, 16)]
        phi = pos_v[pl.ds(l * D + 16, 16)]

        @pl.loop(0, BW, unroll=8)
        def _per_b(b):
            bsplat = jnp.zeros((16,), jnp.int32) + b
            v0 = rows_v[b, pl.ds(0, 16)] + plo
            v1 = rows_v[b, pl.ds(16, 16)] + phi
            plsc.store_scatter(tb_v, [iota, bsplat], v0)
            plsc.store_scatter(tb_v, [dhi, bsplat], v1)

    pltpu.sync_copy(idx_src(0), idx0)
    pltpu.async_copy(idx_src(1), idx1, isem1)
    pltpu.async_copy(tok_hbm.at[idx0], rows0, gsem0)

    @pl.loop(0, S // 2)
    def _pair(k):
        for par in range(2):
            l = 2 * k + par
            c_idx, c_rows, c_tb, c_isem, c_gsem, c_osem = bufs[par]
            n_idx, n_rows, n_tb, n_isem, n_gsem, n_osem = bufs[1 - par]

            @pl.when(l + 1 < S)
            def _fire_next():
                pltpu.make_async_copy(idx_src(l + 1), n_idx, n_isem).wait()
                pltpu.async_copy(tok_hbm.at[n_idx], n_rows, n_gsem)

            pltpu.make_async_copy(tok_hbm.at[c_idx], c_rows, c_gsem).wait()

            @pl.when(l + 2 < S)
            def _prefetch_idx():
                pltpu.async_copy(idx_src(l + 2), c_idx, c_isem)

            @pl.when(l >= 2)
            def _wait_prev_scatter():
                pltpu.make_async_copy(
                    c_tb.at[:, pl.ds(0, BW)], out_dst(l - 2), c_osem).wait()

            transpose_add(l, c_rows, c_tb)
            pltpu.async_copy(c_tb.at[:, pl.ds(0, BW)], out_dst(l), c_osem)

    pltpu.make_async_copy(tb0.at[:, pl.ds(0, BW)], out_dst(S - 2), osem0).wait()
    pltpu.make_async_copy(tb1.at[:, pl.ds(0, BW)], out_dst(S - 1), osem1).wait()


@jax.jit
def _embed(xt, token_table, pos_flat):
    mesh = plsc.VectorSubcoreMesh(core_axis_name="c", subcore_axis_name="s")
    return pl.kernel(
        _embed_body,
        out_type=jax.ShapeDtypeStruct((S, D, B), jnp.float32),
        mesh=mesh,
        compiler_params=pltpu.CompilerParams(
            use_tc_tiling_on_sc=False, needs_layout_passes=False),
        scratch_types=[
            pltpu.VMEM((BW,), jnp.int32),
            pltpu.VMEM((BW,), jnp.int32),
            pltpu.VMEM((BW, D), jnp.float32),
            pltpu.VMEM((BW, D), jnp.float32),
            pltpu.VMEM((D, BWP), jnp.float32),
            pltpu.VMEM((D, BWP), jnp.float32),
            pltpu.VMEM((S * D,), jnp.float32),
            pltpu.SemaphoreType.DMA,
            pltpu.SemaphoreType.DMA,
            pltpu.SemaphoreType.DMA,
            pltpu.SemaphoreType.DMA,
            pltpu.SemaphoreType.DMA,
            pltpu.SemaphoreType.DMA,
        ],
    )(xt, token_table, pos_flat)


def kernel(x, token_table, position_table):
    xt = x.T.astype(jnp.int32)
    pos_flat = position_table[:S].reshape(-1)
    out_sdb = _embed(xt, token_table, pos_flat)
    return jnp.transpose(out_sdb, (2, 0, 1))
